# baseline (speedup 1.0000x reference)
import jax
import jax.numpy as jnp
from jax.experimental import pallas as pl
from jax.experimental.pallas import tpu as pltpu

N = 4096
BS = 256
BM = 1024
L1_STEPS = N // BS
L2_END = L1_STEPS + N // BM

bf = jnp.bfloat16
f32 = jnp.float32


def _split_dot(v, w_ref):
    hi = v.astype(bf)
    lo = (v - hi.astype(f32)).astype(bf)
    w = w_ref[...]
    return (jnp.dot(hi, w, preferred_element_type=f32)
            + jnp.dot(lo, w, preferred_element_type=f32))


def _body(adj_ref, x_ref, w1_ref, b1_ref, w2_ref, b2_ref, w3_ref, b3_ref,
          f1w_ref, f1b_ref, f2w_ref, f2b_ref, f3w_ref, f3b_ref,
          out_ref, adj_bf, u2, u3):
    g = pl.program_id(0)

    @pl.when(g < L1_STEPS)
    def _layer1():
        rows = pl.ds(g * BS, BS)
        blk = adj_ref[...]
        a1 = jnp.dot(blk, x_ref[...], preferred_element_type=f32)
        adj_bf[rows, :] = blk.astype(bf)
        h1 = jnp.tanh(_split_dot(a1, w1_ref) + b1_ref[...])
        u2[rows, :] = jnp.dot(h1.astype(bf), w2_ref[...],
                              preferred_element_type=f32).astype(bf)

    @pl.when((g >= L1_STEPS) & (g < L2_END))
    def _layer2():
        rows = pl.ds((g - L1_STEPS) * BM, BM)
        a2 = jnp.dot(adj_bf[rows, :], u2[...], preferred_element_type=f32)
        h2 = jnp.tanh(a2 + b2_ref[...])
        u3[rows, :] = jnp.dot(h2.astype(bf), w3_ref[...],
                              preferred_element_type=f32).astype(bf)

    @pl.when(g >= L2_END)
    def _layer3_head():
        rows = pl.ds((g - L2_END) * BM, BM)
        h3 = jnp.dot(adj_bf[rows, :], u3[...],
                     preferred_element_type=f32) + b3_ref[...]
        a = jnp.maximum(_split_dot(h3, f1w_ref) + f1b_ref[...], 0.0)
        a = jnp.maximum(_split_dot(a, f2w_ref) + f2b_ref[...], 0.0)
        out_ref[...] = _split_dot(a, f3w_ref) + f3b_ref[...]


def _full(shape):
    return pl.BlockSpec(shape, lambda g: (0,) * len(shape))


def kernel(x, adj, W1, b1, W2, b2, W3, b3,
           fc1_w, fc1_b, fc2_w, fc2_b, fc3_w, fc3_b):
    f1w = jnp.zeros((128, 256), f32).at[:, :152].set(fc1_w.T).astype(bf)
    f1b = jnp.zeros((1, 256), f32).at[0, :152].set(fc1_b)
    f2w = jnp.zeros((256, 128), f32).at[:152, :48].set(fc2_w.T).astype(bf)
    f2b = jnp.zeros((1, 128), f32).at[0, :48].set(fc2_b)
    f3w = jnp.zeros((128, 128), f32).at[:48, :1].set(fc3_w.T).astype(bf)
    f3b = jnp.zeros((1, 128), f32).at[0, :1].set(fc3_b)

    adj_stream = pl.BlockSpec(
        (BS, N), lambda g: (jnp.where(g < L1_STEPS, g, 0), 0))
    out = pl.pallas_call(
        _body,
        grid=(L2_END + N // BM,),
        in_specs=[adj_stream, _full((N, 512)),
                  _full((512, 1024)), _full((1, 1024)),
                  _full((1024, 512)), _full((1, 512)),
                  _full((512, 128)), _full((1, 128)),
                  _full((128, 256)), _full((1, 256)),
                  _full((256, 128)), _full((1, 128)),
                  _full((128, 128)), _full((1, 128))],
        out_specs=pl.BlockSpec(
            (BM, 128), lambda g: (jnp.where(g >= L2_END, g - L2_END, 0), 0)),
        out_shape=jax.ShapeDtypeStruct((N, 128), f32),
        scratch_shapes=[pltpu.VMEM((N, N), bf),
                        pltpu.VMEM((N, 512), bf),
                        pltpu.VMEM((N, 128), bf)],
        compiler_params=pltpu.CompilerParams(
            dimension_semantics=("arbitrary",),
            vmem_limit_bytes=100 * 1024 * 1024,
        ),
    )(adj, x, W1.astype(bf), b1.reshape(1, -1), W2.astype(bf),
      b2.reshape(1, -1), W3.astype(bf), b3.reshape(1, -1),
      f1w, f1b, f2w, f2b, f3w, f3b)
    return out[:, :1]


# speedup vs baseline: 1.1477x; 1.1477x over previous
import jax
import jax.numpy as jnp
from jax.experimental import pallas as pl
from jax.experimental.pallas import tpu as pltpu

N = 4096
BS = 256
BM = 1024
L1_STEPS = N // BS
L2_END = L1_STEPS + N // BM


def _body(adj_ref, x_ref, w1_ref, b1_ref, w2_ref, b2_ref, w3_ref, b3_ref,
          f1w_ref, f1b_ref, f2w_ref, f2b_ref, f3w_ref, f3b_ref,
          out_ref, adj_bf, u2, u3):
    g = pl.program_id(0)
    bf = jnp.bfloat16

    @pl.when(g < L1_STEPS)
    def _layer1():
        rows = pl.ds(g * BS, BS)
        blk = adj_ref[...]
        a1 = jnp.dot(blk, x_ref[...], preferred_element_type=jnp.float32)
        adj_bf[rows, :] = blk.astype(bf)
        h1 = jnp.tanh(jnp.dot(a1, w1_ref[...],
                              preferred_element_type=jnp.float32) + b1_ref[...])
        u2[rows, :] = jnp.dot(h1, w2_ref[...],
                              preferred_element_type=jnp.float32).astype(bf)

    @pl.when((g >= L1_STEPS) & (g < L2_END))
    def _layer2():
        rows = pl.ds((g - L1_STEPS) * BM, BM)
        a2 = jnp.dot(adj_bf[rows, :], u2[...],
                     preferred_element_type=jnp.float32)
        h2 = jnp.tanh(a2 + b2_ref[...])
        u3[rows, :] = jnp.dot(h2, w3_ref[...],
                              preferred_element_type=jnp.float32).astype(bf)

    @pl.when(g >= L2_END)
    def _layer3_head():
        rows = pl.ds((g - L2_END) * BM, BM)
        h3 = jnp.dot(adj_bf[rows, :], u3[...],
                     preferred_element_type=jnp.float32) + b3_ref[...]
        a = jnp.maximum(
            jnp.dot(h3, f1w_ref[...], preferred_element_type=jnp.float32)
            + f1b_ref[...], 0.0)
        a = jnp.maximum(
            jnp.dot(a, f2w_ref[...], preferred_element_type=jnp.float32)
            + f2b_ref[...], 0.0)
        out_ref[...] = (jnp.dot(a, f3w_ref[...],
                                preferred_element_type=jnp.float32)
                        + f3b_ref[...])


def _full(shape):
    return pl.BlockSpec(shape, lambda g: (0,) * len(shape))


def kernel(x, adj, W1, b1, W2, b2, W3, b3,
           fc1_w, fc1_b, fc2_w, fc2_b, fc3_w, fc3_b):
    bf = jnp.bfloat16
    f1w = jnp.zeros((128, 256), jnp.float32).at[:, :152].set(fc1_w.T)
    f1b = jnp.zeros((1, 256), jnp.float32).at[0, :152].set(fc1_b)
    f2w = jnp.zeros((256, 128), jnp.float32).at[:152, :48].set(fc2_w.T)
    f2b = jnp.zeros((1, 128), jnp.float32).at[0, :48].set(fc2_b)
    f3w = jnp.zeros((128, 128), jnp.float32).at[:48, :1].set(fc3_w.T)
    f3b = jnp.zeros((1, 128), jnp.float32).at[0, :1].set(fc3_b)

    adj_stream = pl.BlockSpec(
        (BS, N), lambda g: (jnp.where(g < L1_STEPS, g, 0), 0))
    out = pl.pallas_call(
        _body,
        grid=(L2_END + N // BM,),
        in_specs=[adj_stream, _full((N, 512)),
                  _full((512, 1024)), _full((1, 1024)),
                  _full((1024, 512)), _full((1, 512)),
                  _full((512, 128)), _full((1, 128)),
                  _full((128, 256)), _full((1, 256)),
                  _full((256, 128)), _full((1, 128)),
                  _full((128, 128)), _full((1, 128))],
        out_specs=pl.BlockSpec(
            (BM, 128), lambda g: (jnp.where(g >= L2_END, g - L2_END, 0), 0)),
        out_shape=jax.ShapeDtypeStruct((N, 128), jnp.float32),
        scratch_shapes=[pltpu.VMEM((N, N), bf),
                        pltpu.VMEM((N, 512), bf),
                        pltpu.VMEM((N, 128), bf)],
        compiler_params=pltpu.CompilerParams(
            dimension_semantics=("arbitrary",),
            vmem_limit_bytes=100 * 1024 * 1024,
        ),
    )(adj, x, W1, b1.reshape(1, -1), W2, b2.reshape(1, -1),
      W3, b3.reshape(1, -1), f1w, f1b, f2w, f2b, f3w, f3b)
    return out[:, :1]
